# baseline (speedup 1.0000x reference)
import functools
import jax
import jax.numpy as jnp
from jax import lax
from jax.experimental import pallas as pl
from jax.experimental.pallas import tpu as pltpu
from jax.experimental.pallas import tpu_sc as plsc

_B, _S, _D = 4, 4096, 1024
_NC, _NS = 2, 16
_NW = _NC * _NS
_ROWS = _B * _S
_RPW = _ROWS // _NW
_CH = 32
_NCHUNK = _RPW // _CH
_CHW = _CH * _D


def _sc_body(x_hbm, pe_hbm, out_hbm, xb, peb, sem_x, sem_pe):
    wid = lax.axis_index("s") * _NC + lax.axis_index("c")
    row0 = wid * _RPW
    s0 = (wid % (_S // _RPW)) * _RPW

    def chunk(g, carry):
        off_x = (row0 + g * _CH) * _D
        off_pe = (s0 + g * _CH) * _D
        cx = pltpu.async_copy(x_hbm.at[pl.ds(off_x, _CHW)], xb, sem_x)
        cp = pltpu.async_copy(pe_hbm.at[pl.ds(off_pe, _CHW)], peb, sem_pe)
        cx.wait()
        cp.wait()

        def add16(j, c):
            sl = pl.ds(j * 16, 16)
            plsc.addupdate(xb.at[sl], peb[sl])
            return c

        lax.fori_loop(0, _CHW // 16, add16, 0, unroll=8)
        pltpu.sync_copy(xb, out_hbm.at[pl.ds(off_x, _CHW)])
        return carry

    lax.fori_loop(0, _NCHUNK, chunk, 0)


def kernel(x, pos_embedding):
    B, S, D = x.shape
    x1 = x.reshape(-1)
    pe1 = pos_embedding.reshape(-1)
    mesh = plsc.VectorSubcoreMesh(core_axis_name="c", subcore_axis_name="s")
    out = pl.kernel(
        _sc_body,
        out_type=jax.ShapeDtypeStruct((B * S * D,), x.dtype),
        mesh=mesh,
        scratch_types=[
            pltpu.VMEM((_CHW,), jnp.float32),
            pltpu.VMEM((_CHW,), jnp.float32),
            pltpu.SemaphoreType.DMA,
            pltpu.SemaphoreType.DMA,
        ],
    )(x1, pe1)
    return out.reshape(B, S, D)


# speedup vs baseline: 1.2088x; 1.2088x over previous
import jax
import jax.numpy as jnp
from jax import lax
from jax.experimental import pallas as pl
from jax.experimental.pallas import tpu as pltpu
from jax.experimental.pallas import tpu_sc as plsc

_B, _S, _D = 4, 4096, 1024
_NC, _NS = 2, 16
_NW = _NC * _NS
_SPW = _S // _NW
_CH = 16
_NCHUNK = _SPW // _CH
_CHW = _CH * _D
_NSTEP = _NCHUNK * _B


def _sc_body(x_hbm, pe_hbm, out_hbm,
             xb0, xb1, pb0, pb1,
             sx0, sx1, sp0, sp1, so0, so1):
    wid = lax.axis_index("s") * _NC + lax.axis_index("c")
    s_base = wid * _SPW
    xbs, pbs = (xb0, xb1), (pb0, pb1)
    sxs, sps, sos = (sx0, sx1), (sp0, sp1), (so0, so1)

    def x_off(t):
        b, g = t % _B, t // _B
        return (b * _S + s_base + g * _CH) * _D

    def fill_x(t):
        pltpu.async_copy(x_hbm.at[pl.ds(x_off(t), _CHW)], xbs[t % 2], sxs[t % 2])

    def fill_pe(g):
        pltpu.async_copy(pe_hbm.at[pl.ds((s_base + g * _CH) * _D, _CHW)],
                         pbs[g % 2], sps[g % 2])

    def wait_in(ref, sem):
        pltpu.make_async_copy(x_hbm.at[pl.ds(0, _CHW)], ref, sem).wait()

    def wait_out(p):
        pltpu.make_async_copy(xbs[p], out_hbm.at[pl.ds(0, _CHW)], sos[p]).wait()

    fill_x(0)
    fill_pe(0)
    for t in range(_NSTEP):
        b, g, p = t % _B, t // _B, t % 2
        if t + 1 < _NSTEP:
            if t >= 1:
                wait_out((t + 1) % 2)
            fill_x(t + 1)
        if b == 0 and g + 1 < _NCHUNK:
            fill_pe(g + 1)
        wait_in(xbs[p], sxs[p])
        if b == 0:
            wait_in(pbs[g % 2], sps[g % 2])
        xr, pr = xbs[p], pbs[g % 2]

        def add16(j, c, xr=xr, pr=pr):
            sl = pl.ds(j * 16, 16)
            plsc.addupdate(xr.at[sl], pr[sl])
            return c

        lax.fori_loop(0, _CHW // 16, add16, 0, unroll=8)
        pltpu.async_copy(xbs[p], out_hbm.at[pl.ds(x_off(t), _CHW)], sos[p])
    wait_out((_NSTEP - 1) % 2)


def kernel(x, pos_embedding):
    B, S, D = x.shape
    x1 = x.reshape(-1)
    pe1 = pos_embedding.reshape(-1)
    mesh = plsc.VectorSubcoreMesh(core_axis_name="c", subcore_axis_name="s")
    out = pl.kernel(
        _sc_body,
        out_type=jax.ShapeDtypeStruct((B * S * D,), x.dtype),
        mesh=mesh,
        scratch_types=[
            pltpu.VMEM((_CHW,), jnp.float32),
            pltpu.VMEM((_CHW,), jnp.float32),
            pltpu.VMEM((_CHW,), jnp.float32),
            pltpu.VMEM((_CHW,), jnp.float32),
            pltpu.SemaphoreType.DMA,
            pltpu.SemaphoreType.DMA,
            pltpu.SemaphoreType.DMA,
            pltpu.SemaphoreType.DMA,
            pltpu.SemaphoreType.DMA,
            pltpu.SemaphoreType.DMA,
        ],
    )(x1, pe1)
    return out.reshape(B, S, D)


# speedup vs baseline: 1.2643x; 1.0459x over previous
import jax
import jax.numpy as jnp
from jax import lax
from jax.experimental import pallas as pl
from jax.experimental.pallas import tpu as pltpu
from jax.experimental.pallas import tpu_sc as plsc

_B, _S, _D = 4, 4096, 1024
_NC, _NS = 2, 16
_NW = _NC * _NS
_SPW = _S // _NW
_CH = 16
_NCHUNK = _SPW // _CH
_CHW = _CH * _D
_NSTEP = _NCHUNK * _B
_XR = 4
_PR = 3


def _sc_body(x_hbm, pe_hbm, out_hbm, *refs):
    xbs = refs[0:_XR]
    pbs = refs[_XR:_XR + _PR]
    sxs = refs[_XR + _PR:2 * _XR + _PR]
    sps = refs[2 * _XR + _PR:2 * _XR + 2 * _PR]
    sos = refs[2 * _XR + 2 * _PR:3 * _XR + 2 * _PR]

    wid = lax.axis_index("s") * _NC + lax.axis_index("c")
    s_base = wid * _SPW

    def x_off(t):
        b, g = t % _B, t // _B
        return (b * _S + s_base + g * _CH) * _D

    def fill_x(t):
        pltpu.async_copy(x_hbm.at[pl.ds(x_off(t), _CHW)], xbs[t % _XR], sxs[t % _XR])

    def fill_pe(g):
        pltpu.async_copy(pe_hbm.at[pl.ds((s_base + g * _CH) * _D, _CHW)],
                         pbs[g % _PR], sps[g % _PR])

    def wait_in(ref, sem):
        pltpu.make_async_copy(x_hbm.at[pl.ds(0, _CHW)], ref, sem).wait()

    def wait_out(p):
        pltpu.make_async_copy(xbs[p], out_hbm.at[pl.ds(0, _CHW)], sos[p]).wait()

    fill_x(0)
    fill_x(1)
    fill_pe(0)
    fill_pe(1)
    for t in range(_NSTEP):
        b, g, p = t % _B, t // _B, t % _XR
        if t + 2 < _NSTEP:
            if t >= 2:
                wait_out((t + 2) % _XR)
            fill_x(t + 2)
        if b == 0 and g + 2 < _NCHUNK:
            fill_pe(g + 2)
        wait_in(xbs[p], sxs[p])
        if b == 0:
            wait_in(pbs[g % _PR], sps[g % _PR])
        xr, pr = xbs[p], pbs[g % _PR]

        @plsc.parallel_loop(0, _CHW, step=16, unroll=8)
        def _add16(i, xr=xr, pr=pr):
            plsc.addupdate(xr.at[pl.ds(i, 16)], pr[pl.ds(i, 16)])

        pltpu.async_copy(xbs[p], out_hbm.at[pl.ds(x_off(t), _CHW)], sos[p])
    for t in range(_NSTEP - 4, _NSTEP):
        wait_out(t % _XR)


def kernel(x, pos_embedding):
    B, S, D = x.shape
    x1 = x.reshape(-1)
    pe1 = pos_embedding.reshape(-1)
    mesh = plsc.VectorSubcoreMesh(core_axis_name="c", subcore_axis_name="s")
    out = pl.kernel(
        _sc_body,
        out_type=jax.ShapeDtypeStruct((B * S * D,), x.dtype),
        mesh=mesh,
        scratch_types=(
            [pltpu.VMEM((_CHW,), jnp.float32)] * (_XR + _PR)
            + [pltpu.SemaphoreType.DMA] * (2 * _XR + 2 * _PR)
        ),
    )(x1, pe1)
    return out.reshape(B, S, D)
